# baseline (speedup 1.0000x reference)
import jax
import jax.numpy as jnp
from jax import lax
from jax.experimental import pallas as pl
from jax.experimental.pallas import tpu as pltpu
from jax.experimental.pallas import tpu_sc as plsc

_N = 10000
_D = 128
_E = 320000
_NPAD = 10240
_NC = 2
_NS = 16
_NW = _NC * _NS
_EB = 64
_NB = 160
_CH = 16
_NCH = _NB // _CH
_EPAD = _NW * _NB * _EB
_RPT = _NPAD // _NS
_HB = _NPAD // 128
_HPT = _HB // _NS
_BLK = 1024

_mesh = plsc.VectorSubcoreMesh(core_axis_name="c", subcore_axis_name="s")


def _sc_degree_body(col_hbm, out_hbm, colv, hist, acc, gbuf, shared):
    c = lax.axis_index("c")
    s = lax.axis_index("s")
    wid = c * _NS + s
    pltpu.sync_copy(col_hbm.at[pl.ds(wid * _NB, _NB)], colv)

    @pl.loop(0, _NPAD // 16)
    def _zero(i):
        hist[pl.ds(i * 16, 16)] = jnp.zeros((16,), jnp.float32)

    ones = jnp.full((16,), 1.0, jnp.float32)

    @pl.loop(0, _NB)
    def _count(b):
        for g in range(_EB // 16):
            e = colv[b, pl.ds(g * 16, 16)]
            plsc.addupdate_scatter(hist, [e], ones)

    pltpu.sync_copy(hist, shared.at[s])
    plsc.subcore_barrier()
    pltpu.sync_copy(shared.at[:, pl.ds(s * _RPT, _RPT)], gbuf)

    @pl.loop(0, _RPT // 16)
    def _sum(i):
        v = gbuf[0, pl.ds(i * 16, 16)]
        for r in range(1, _NS):
            v = v + gbuf[r, pl.ds(i * 16, 16)]
        acc[pl.ds(i * 16, 16)] = v

    pltpu.sync_copy(acc, out_hbm.at[pl.ds(c * _NPAD + s * _RPT, _RPT)])


_deg_call = pl.kernel(
    _sc_degree_body,
    out_type=jax.ShapeDtypeStruct((_NC * _NPAD,), jnp.float32),
    mesh=_mesh,
    compiler_params=pltpu.CompilerParams(needs_layout_passes=False),
    scratch_types=[
        pltpu.VMEM((_NB, _EB), jnp.int32),
        pltpu.VMEM((_NPAD,), jnp.float32),
        pltpu.VMEM((_RPT,), jnp.float32),
        pltpu.VMEM((_NS, _RPT), jnp.float32),
        pltpu.VMEM_SHARED((_NS, _NPAD), jnp.float32),
    ],
)


def _sc_scatter_body(y_hbm, row_hbm, col_hbm, zeros_hbm, out_hbm,
                     rowv, colv, rbuf0, rbuf1, rbuf2, rbuf3, shared,
                     sem0, sem1, sem2, sem3):
    c = lax.axis_index("c")
    s = lax.axis_index("s")
    wid = c * _NS + s
    pltpu.sync_copy(zeros_hbm, shared.at[pl.ds(s * _RPT, _RPT)])
    plsc.subcore_barrier()

    @pl.loop(0, _NCH)
    def _chunk(ch):
        pltpu.sync_copy(row_hbm.at[pl.ds(wid * _NB + ch * _CH, _CH)], rowv)
        pltpu.sync_copy(col_hbm.at[pl.ds(wid * _NB + ch * _CH, _CH)], colv)

        @pl.loop(0, _CH // 4)
        def _quad(i):
            b = i * 4
            cp0 = pltpu.async_copy(y_hbm.at[rowv.at[b]], rbuf0, sem0)
            cp1 = pltpu.async_copy(y_hbm.at[rowv.at[b + 1]], rbuf1, sem1)
            cp2 = pltpu.async_copy(y_hbm.at[rowv.at[b + 2]], rbuf2, sem2)
            cp3 = pltpu.async_copy(y_hbm.at[rowv.at[b + 3]], rbuf3, sem3)
            cp0.wait()
            pltpu.sync_copy(rbuf0, shared.at[colv.at[b]], add=True)
            cp1.wait()
            pltpu.sync_copy(rbuf1, shared.at[colv.at[b + 1]], add=True)
            cp2.wait()
            pltpu.sync_copy(rbuf2, shared.at[colv.at[b + 2]], add=True)
            cp3.wait()
            pltpu.sync_copy(rbuf3, shared.at[colv.at[b + 3]], add=True)

    plsc.subcore_barrier()
    pltpu.sync_copy(shared.at[pl.ds(s * _RPT, _RPT)],
                    out_hbm.at[c, pl.ds(s * _RPT, _RPT)])


_scatter_call = pl.kernel(
    _sc_scatter_body,
    out_type=jax.ShapeDtypeStruct((_NC, _NPAD, _D), jnp.float32),
    mesh=_mesh,
    compiler_params=pltpu.CompilerParams(needs_layout_passes=False),
    scratch_types=[
        pltpu.VMEM((_CH, _EB), jnp.int32),
        pltpu.VMEM((_CH, _EB), jnp.int32),
        pltpu.VMEM((_EB, _D), jnp.float32),
        pltpu.VMEM((_EB, _D), jnp.float32),
        pltpu.VMEM((_EB, _D), jnp.float32),
        pltpu.VMEM((_EB, _D), jnp.float32),
        pltpu.VMEM_SHARED((_NPAD, _D), jnp.float32),
        pltpu.SemaphoreType.DMA,
        pltpu.SemaphoreType.DMA,
        pltpu.SemaphoreType.DMA,
        pltpu.SemaphoreType.DMA,
    ],
)


def _scales(ha_ref, hb_ref):
    deg = ha_ref[...] + hb_ref[...] + 1.0
    return lax.rsqrt(deg), 1.0 / deg


def _tc_pre_body(x_ref, wt_ref, bt_ref, ha_ref, hb_ref, y_ref, t_ref):
    dis, dinv = _scales(ha_ref, hb_ref)
    x = x_ref[...]
    lin = jnp.dot(x, wt_ref[...], preferred_element_type=jnp.float32)
    xb = jnp.dot(x, bt_ref[...], preferred_element_type=jnp.float32)
    y_ref[...] = dis * lin
    t_ref[...] = dinv * lin - xb


def _tc_mid_body(sa_ref, sb_ref, t0_ref, ha_ref, hb_ref, wt_ref, bt_ref,
                 y_ref, t_ref):
    dis, dinv = _scales(ha_ref, hb_ref)
    h = jnp.maximum(dis * (sa_ref[...] + sb_ref[...]) + t0_ref[...], 0.0)
    lin = jnp.dot(h, wt_ref[...], preferred_element_type=jnp.float32)
    xb = jnp.dot(h, bt_ref[...], preferred_element_type=jnp.float32)
    y_ref[...] = dis * lin
    t_ref[...] = dinv * lin - xb


def _tc_fin_body(sa_ref, sb_ref, t1_ref, ha_ref, hb_ref, o_ref):
    dis, _ = _scales(ha_ref, hb_ref)
    o_ref[...] = jnp.maximum(
        dis * (sa_ref[...] + sb_ref[...]) + t1_ref[...], 0.0)


_row = pl.BlockSpec((_BLK, _D), lambda i: (i, 0))
_colv = pl.BlockSpec((_BLK, 1), lambda i: (i, 0))
_wsp = pl.BlockSpec((_D, _D), lambda i: (0, 0))
_grid = (_NPAD // _BLK,)
_fout = jax.ShapeDtypeStruct((_NPAD, _D), jnp.float32)

_tc_pre = pl.pallas_call(
    _tc_pre_body,
    grid=_grid,
    in_specs=[_row, _wsp, _wsp, _colv, _colv],
    out_specs=(_row, _row),
    out_shape=(_fout, _fout),
)

_tc_mid = pl.pallas_call(
    _tc_mid_body,
    grid=_grid,
    in_specs=[_row, _row, _row, _colv, _colv, _wsp, _wsp],
    out_specs=(_row, _row),
    out_shape=(_fout, _fout),
)

_tc_fin = pl.pallas_call(
    _tc_fin_body,
    grid=_grid,
    in_specs=[_row, _row, _row, _colv, _colv],
    out_specs=_row,
    out_shape=_fout,
)


def kernel(x, edge_index, W0, B0, W1, B1):
    xp = jnp.pad(x, ((0, _NPAD - _N), (0, 0)))
    row = jnp.pad(edge_index[0], (0, _EPAD - _E),
                  constant_values=_N).reshape(_NW * _NB, _EB)
    col = jnp.pad(edge_index[1], (0, _EPAD - _E),
                  constant_values=_N).reshape(_NW * _NB, _EB)
    zeros = jnp.zeros((_RPT, _D), jnp.float32)

    hist = _deg_call(col)
    ha = hist[:_NPAD].reshape(_NPAD, 1)
    hb = hist[_NPAD:].reshape(_NPAD, 1)

    y0, t0 = _tc_pre(xp, W0.T, B0.T, ha, hb)
    s0 = _scatter_call(y0, row, col, zeros)
    y1, t1 = _tc_mid(s0[0], s0[1], t0, ha, hb, W1.T, B1.T)
    s1 = _scatter_call(y1, row, col, zeros)
    out = _tc_fin(s1[0], s1[1], t1, ha, hb)
    return out[:_N]


# speedup vs baseline: 1.2921x; 1.2921x over previous
import jax
import jax.numpy as jnp
from jax import lax
from jax.experimental import pallas as pl
from jax.experimental.pallas import tpu as pltpu
from jax.experimental.pallas import tpu_sc as plsc

_N = 10000
_D = 128
_E = 320000
_NPAD = 10240
_NC = 2
_NS = 16
_NW = _NC * _NS
_EB = 128
_NB = 80
_CH = 16
_NCH = _NB // _CH
_EPAD = _NW * _NB * _EB
_RPT = _NPAD // _NS
_HB = _NPAD // 128
_HPT = _HB // _NS
_BLK = 1024

_mesh = plsc.VectorSubcoreMesh(core_axis_name="c", subcore_axis_name="s")


def _sc_degree_body(col_hbm, out_hbm, colv, hist, acc, gbuf, shared):
    c = lax.axis_index("c")
    s = lax.axis_index("s")
    wid = c * _NS + s
    pltpu.sync_copy(col_hbm.at[pl.ds(wid * _NB, _NB)], colv)

    @pl.loop(0, _NPAD // 16)
    def _zero(i):
        hist[pl.ds(i * 16, 16)] = jnp.zeros((16,), jnp.float32)

    ones = jnp.full((16,), 1.0, jnp.float32)

    @pl.loop(0, _NB)
    def _count(b):
        for g in range(_EB // 16):
            e = colv[b, pl.ds(g * 16, 16)]
            plsc.addupdate_scatter(hist, [e], ones)

    pltpu.sync_copy(hist, shared.at[s])
    plsc.subcore_barrier()
    pltpu.sync_copy(shared.at[:, pl.ds(s * _RPT, _RPT)], gbuf)

    @pl.loop(0, _RPT // 16)
    def _sum(i):
        v = gbuf[0, pl.ds(i * 16, 16)]
        for r in range(1, _NS):
            v = v + gbuf[r, pl.ds(i * 16, 16)]
        acc[pl.ds(i * 16, 16)] = v

    pltpu.sync_copy(acc, out_hbm.at[pl.ds(c * _NPAD + s * _RPT, _RPT)])


_deg_call = pl.kernel(
    _sc_degree_body,
    out_type=jax.ShapeDtypeStruct((_NC * _NPAD,), jnp.float32),
    mesh=_mesh,
    compiler_params=pltpu.CompilerParams(needs_layout_passes=False),
    scratch_types=[
        pltpu.VMEM((_NB, _EB), jnp.int32),
        pltpu.VMEM((_NPAD,), jnp.float32),
        pltpu.VMEM((_RPT,), jnp.float32),
        pltpu.VMEM((_NS, _RPT), jnp.float32),
        pltpu.VMEM_SHARED((_NS, _NPAD), jnp.float32),
    ],
)


def _sc_scatter_body(y_hbm, row_hbm, col_hbm, zeros_hbm, out_hbm,
                     rowv, colv, rbuf0, rbuf1, shared, sem0, sem1,
                     sems0, sems1):
    c = lax.axis_index("c")
    s = lax.axis_index("s")
    wid = c * _NS + s
    pltpu.sync_copy(zeros_hbm, shared.at[pl.ds(s * _RPT, _RPT)])
    plsc.subcore_barrier()

    @pl.loop(0, _NCH)
    def _chunk(ch):
        pltpu.sync_copy(row_hbm.at[pl.ds(wid * _NB + ch * _CH, _CH)], rowv)
        pltpu.sync_copy(col_hbm.at[pl.ds(wid * _NB + ch * _CH, _CH)], colv)

        rbufs = (rbuf0, rbuf1)
        gsems = (sem0, sem1)
        ssems = (sems0, sems1)
        gat = {}
        sca = {}
        for b in range(_CH):
            j = b & 1
            if b >= 2:
                sca[b - 2].wait()
            gat[b] = pltpu.async_copy(y_hbm.at[rowv.at[b]], rbufs[j],
                                      gsems[j])
            if b >= 1:
                gat[b - 1].wait()
                sca[b - 1] = pltpu.async_copy(
                    rbufs[1 - j], shared.at[colv.at[b - 1]], ssems[1 - j],
                    add=True)
        gat[_CH - 1].wait()
        sca[_CH - 1] = pltpu.async_copy(
            rbufs[(_CH - 1) & 1], shared.at[colv.at[_CH - 1]],
            ssems[(_CH - 1) & 1], add=True)
        sca[_CH - 2].wait()
        sca[_CH - 1].wait()

    plsc.subcore_barrier()
    pltpu.sync_copy(shared.at[pl.ds(s * _RPT, _RPT)],
                    out_hbm.at[c, pl.ds(s * _RPT, _RPT)])


_scatter_call = pl.kernel(
    _sc_scatter_body,
    out_type=jax.ShapeDtypeStruct((_NC, _NPAD, _D), jnp.float32),
    mesh=_mesh,
    compiler_params=pltpu.CompilerParams(needs_layout_passes=False),
    scratch_types=[
        pltpu.VMEM((_CH, _EB), jnp.int32),
        pltpu.VMEM((_CH, _EB), jnp.int32),
        pltpu.VMEM((_EB, _D), jnp.float32),
        pltpu.VMEM((_EB, _D), jnp.float32),
        pltpu.VMEM_SHARED((_NPAD, _D), jnp.float32),
        pltpu.SemaphoreType.DMA,
        pltpu.SemaphoreType.DMA,
        pltpu.SemaphoreType.DMA,
        pltpu.SemaphoreType.DMA,
    ],
)


def _scales(ha_ref, hb_ref):
    deg = ha_ref[...] + hb_ref[...] + 1.0
    return lax.rsqrt(deg), 1.0 / deg


def _tc_pre_body(x_ref, wt_ref, bt_ref, ha_ref, hb_ref, y_ref, t_ref):
    dis, dinv = _scales(ha_ref, hb_ref)
    x = x_ref[...]
    lin = jnp.dot(x, wt_ref[...], preferred_element_type=jnp.float32)
    xb = jnp.dot(x, bt_ref[...], preferred_element_type=jnp.float32)
    y_ref[...] = dis * lin
    t_ref[...] = dinv * lin - xb


def _tc_mid_body(sa_ref, sb_ref, t0_ref, ha_ref, hb_ref, wt_ref, bt_ref,
                 y_ref, t_ref):
    dis, dinv = _scales(ha_ref, hb_ref)
    h = jnp.maximum(dis * (sa_ref[...] + sb_ref[...]) + t0_ref[...], 0.0)
    lin = jnp.dot(h, wt_ref[...], preferred_element_type=jnp.float32)
    xb = jnp.dot(h, bt_ref[...], preferred_element_type=jnp.float32)
    y_ref[...] = dis * lin
    t_ref[...] = dinv * lin - xb


def _tc_fin_body(sa_ref, sb_ref, t1_ref, ha_ref, hb_ref, o_ref):
    dis, _ = _scales(ha_ref, hb_ref)
    o_ref[...] = jnp.maximum(
        dis * (sa_ref[...] + sb_ref[...]) + t1_ref[...], 0.0)


_row = pl.BlockSpec((_BLK, _D), lambda i: (i, 0))
_colv = pl.BlockSpec((_BLK, 1), lambda i: (i, 0))
_wsp = pl.BlockSpec((_D, _D), lambda i: (0, 0))
_grid = (_NPAD // _BLK,)
_fout = jax.ShapeDtypeStruct((_NPAD, _D), jnp.float32)

_tc_pre = pl.pallas_call(
    _tc_pre_body,
    grid=_grid,
    in_specs=[_row, _wsp, _wsp, _colv, _colv],
    out_specs=(_row, _row),
    out_shape=(_fout, _fout),
)

_tc_mid = pl.pallas_call(
    _tc_mid_body,
    grid=_grid,
    in_specs=[_row, _row, _row, _colv, _colv, _wsp, _wsp],
    out_specs=(_row, _row),
    out_shape=(_fout, _fout),
)

_tc_fin = pl.pallas_call(
    _tc_fin_body,
    grid=_grid,
    in_specs=[_row, _row, _row, _colv, _colv],
    out_specs=_row,
    out_shape=_fout,
)


def kernel(x, edge_index, W0, B0, W1, B1):
    xp = jnp.pad(x, ((0, _NPAD - _N), (0, 0)))
    row = jnp.pad(edge_index[0], (0, _EPAD - _E),
                  constant_values=_N).reshape(_NW * _NB, _EB)
    col = jnp.pad(edge_index[1], (0, _EPAD - _E),
                  constant_values=_N).reshape(_NW * _NB, _EB)
    zeros = jnp.zeros((_RPT, _D), jnp.float32)

    hist = _deg_call(col)
    ha = hist[:_NPAD].reshape(_NPAD, 1)
    hb = hist[_NPAD:].reshape(_NPAD, 1)

    y0, t0 = _tc_pre(xp, W0.T, B0.T, ha, hb)
    s0 = _scatter_call(y0, row, col, zeros)
    y1, t1 = _tc_mid(s0[0], s0[1], t0, ha, hb, W1.T, B1.T)
    s1 = _scatter_call(y1, row, col, zeros)
    out = _tc_fin(s1[0], s1[1], t1, ha, hb)
    return out[:_N]


# speedup vs baseline: 1.3167x; 1.0190x over previous
import jax
import jax.numpy as jnp
from jax import lax
from jax.experimental import pallas as pl
from jax.experimental.pallas import tpu as pltpu
from jax.experimental.pallas import tpu_sc as plsc

_N = 10000
_D = 128
_E = 320000
_NPAD = 10240
_NC = 2
_NS = 16
_NW = _NC * _NS
_EB = 128
_NB = 80
_CH = 16
_NCH = _NB // _CH
_EPAD = _NW * _NB * _EB
_RPT = _NPAD // _NS
_HB = _NPAD // 128
_HPT = _HB // _NS
_BLK = 1024

_mesh = plsc.VectorSubcoreMesh(core_axis_name="c", subcore_axis_name="s")


def _sc_degree_body(col_hbm, out_hbm, colv, hist, acc, gbuf, shared):
    c = lax.axis_index("c")
    s = lax.axis_index("s")
    wid = c * _NS + s
    pltpu.sync_copy(col_hbm.at[pl.ds(wid * _NB, _NB)], colv)

    @pl.loop(0, _NPAD // 16)
    def _zero(i):
        hist[pl.ds(i * 16, 16)] = jnp.zeros((16,), jnp.float32)

    ones = jnp.full((16,), 1.0, jnp.float32)

    @pl.loop(0, _NB)
    def _count(b):
        for g in range(_EB // 16):
            e = colv[b, pl.ds(g * 16, 16)]
            plsc.addupdate_scatter(hist, [e], ones)

    pltpu.sync_copy(hist, shared.at[s])
    plsc.subcore_barrier()
    pltpu.sync_copy(shared.at[:, pl.ds(s * _RPT, _RPT)], gbuf)

    @pl.loop(0, _RPT // 16)
    def _sum(i):
        v = gbuf[0, pl.ds(i * 16, 16)]
        for r in range(1, _NS):
            v = v + gbuf[r, pl.ds(i * 16, 16)]
        acc[pl.ds(i * 16, 16)] = v

    pltpu.sync_copy(acc, out_hbm.at[pl.ds(c * _NPAD + s * _RPT, _RPT)])


_deg_call = pl.kernel(
    _sc_degree_body,
    out_type=jax.ShapeDtypeStruct((_NC * _NPAD,), jnp.float32),
    mesh=_mesh,
    compiler_params=pltpu.CompilerParams(needs_layout_passes=False),
    scratch_types=[
        pltpu.VMEM((_NB, _EB), jnp.int32),
        pltpu.VMEM((_NPAD,), jnp.float32),
        pltpu.VMEM((_RPT,), jnp.float32),
        pltpu.VMEM((_NS, _RPT), jnp.float32),
        pltpu.VMEM_SHARED((_NS, _NPAD), jnp.float32),
    ],
)


def _sc_scatter_body(y_hbm, row_hbm, col_hbm, zeros_hbm, out_hbm,
                     rowv0, colv0, rowv1, colv1, rbuf0, rbuf1, shared,
                     sem0, sem1, sems0, sems1, isem0, isem1):
    c = lax.axis_index("c")
    s = lax.axis_index("s")
    wid = c * _NS + s
    pltpu.sync_copy(zeros_hbm, shared.at[pl.ds(s * _RPT, _RPT)])
    pltpu.sync_copy(row_hbm.at[pl.ds(wid * _NB, _CH)], rowv0)
    pltpu.sync_copy(col_hbm.at[pl.ds(wid * _NB, _CH)], colv0)
    plsc.subcore_barrier()

    idxbufs = ((rowv0, colv0), (rowv1, colv1))
    rbufs = (rbuf0, rbuf1)
    gsems = (sem0, sem1)
    ssems = (sems0, sems1)
    gat, sca, ipf = {}, {}, {}

    def _wait_sca(g):
        d = sca.pop(g, None)
        if d is not None:
            d.wait()

    for ch in range(_NCH):
        rv, cv = idxbufs[ch & 1]
        if ch > 0:
            _wait_sca(ch * _CH - 2)
            _wait_sca(ch * _CH - 1)
            ipf[ch][0].wait()
            ipf[ch][1].wait()
        if ch + 1 < _NCH:
            nrv, ncv = idxbufs[(ch + 1) & 1]
            base = wid * _NB + (ch + 1) * _CH
            ipf[ch + 1] = (
                pltpu.async_copy(row_hbm.at[pl.ds(base, _CH)], nrv, isem0),
                pltpu.async_copy(col_hbm.at[pl.ds(base, _CH)], ncv, isem1),
            )
        for b in range(_CH):
            g = ch * _CH + b
            j = g & 1
            if g >= 2:
                _wait_sca(g - 2)
            gat[g] = pltpu.async_copy(y_hbm.at[rv.at[b]], rbufs[j],
                                      gsems[j])
            if g >= 1:
                pcv = cv if b >= 1 else idxbufs[(ch - 1) & 1][1]
                pb = b - 1 if b >= 1 else _CH - 1
                gat.pop(g - 1).wait()
                sca[g - 1] = pltpu.async_copy(
                    rbufs[1 - j], shared.at[pcv.at[pb]], ssems[1 - j],
                    add=True)
    glast = _NB - 1
    gat.pop(glast).wait()
    sca[glast] = pltpu.async_copy(
        rbufs[glast & 1], shared.at[idxbufs[(_NCH - 1) & 1][1].at[_CH - 1]],
        ssems[glast & 1], add=True)
    _wait_sca(glast - 1)
    _wait_sca(glast)

    plsc.subcore_barrier()
    pltpu.sync_copy(shared.at[pl.ds(s * _RPT, _RPT)],
                    out_hbm.at[c, pl.ds(s * _RPT, _RPT)])


_scatter_call = pl.kernel(
    _sc_scatter_body,
    out_type=jax.ShapeDtypeStruct((_NC, _NPAD, _D), jnp.float32),
    mesh=_mesh,
    compiler_params=pltpu.CompilerParams(needs_layout_passes=False),
    scratch_types=[
        pltpu.VMEM((_CH, _EB), jnp.int32),
        pltpu.VMEM((_CH, _EB), jnp.int32),
        pltpu.VMEM((_CH, _EB), jnp.int32),
        pltpu.VMEM((_CH, _EB), jnp.int32),
        pltpu.VMEM((_EB, _D), jnp.float32),
        pltpu.VMEM((_EB, _D), jnp.float32),
        pltpu.VMEM_SHARED((_NPAD, _D), jnp.float32),
        pltpu.SemaphoreType.DMA,
        pltpu.SemaphoreType.DMA,
        pltpu.SemaphoreType.DMA,
        pltpu.SemaphoreType.DMA,
        pltpu.SemaphoreType.DMA,
        pltpu.SemaphoreType.DMA,
    ],
)


def _scales(ha_ref, hb_ref):
    deg = ha_ref[...] + hb_ref[...] + 1.0
    return lax.rsqrt(deg), 1.0 / deg


def _tc_pre_body(x_ref, wt_ref, bt_ref, ha_ref, hb_ref, y_ref, t_ref):
    dis, dinv = _scales(ha_ref, hb_ref)
    x = x_ref[...]
    lin = jnp.dot(x, wt_ref[...], preferred_element_type=jnp.float32)
    xb = jnp.dot(x, bt_ref[...], preferred_element_type=jnp.float32)
    y_ref[...] = dis * lin
    t_ref[...] = dinv * lin - xb


def _tc_mid_body(sa_ref, sb_ref, t0_ref, ha_ref, hb_ref, wt_ref, bt_ref,
                 y_ref, t_ref):
    dis, dinv = _scales(ha_ref, hb_ref)
    h = jnp.maximum(dis * (sa_ref[...] + sb_ref[...]) + t0_ref[...], 0.0)
    lin = jnp.dot(h, wt_ref[...], preferred_element_type=jnp.float32)
    xb = jnp.dot(h, bt_ref[...], preferred_element_type=jnp.float32)
    y_ref[...] = dis * lin
    t_ref[...] = dinv * lin - xb


def _tc_fin_body(sa_ref, sb_ref, t1_ref, ha_ref, hb_ref, o_ref):
    dis, _ = _scales(ha_ref, hb_ref)
    o_ref[...] = jnp.maximum(
        dis * (sa_ref[...] + sb_ref[...]) + t1_ref[...], 0.0)


_row = pl.BlockSpec((_BLK, _D), lambda i: (i, 0))
_colv = pl.BlockSpec((_BLK, 1), lambda i: (i, 0))
_wsp = pl.BlockSpec((_D, _D), lambda i: (0, 0))
_grid = (_NPAD // _BLK,)
_fout = jax.ShapeDtypeStruct((_NPAD, _D), jnp.float32)

_tc_pre = pl.pallas_call(
    _tc_pre_body,
    grid=_grid,
    in_specs=[_row, _wsp, _wsp, _colv, _colv],
    out_specs=(_row, _row),
    out_shape=(_fout, _fout),
)

_tc_mid = pl.pallas_call(
    _tc_mid_body,
    grid=_grid,
    in_specs=[_row, _row, _row, _colv, _colv, _wsp, _wsp],
    out_specs=(_row, _row),
    out_shape=(_fout, _fout),
)

_tc_fin = pl.pallas_call(
    _tc_fin_body,
    grid=_grid,
    in_specs=[_row, _row, _row, _colv, _colv],
    out_specs=_row,
    out_shape=_fout,
)


def kernel(x, edge_index, W0, B0, W1, B1):
    xp = jnp.pad(x, ((0, _NPAD - _N), (0, 0)))
    row = jnp.pad(edge_index[0], (0, _EPAD - _E),
                  constant_values=_N).reshape(_NW * _NB, _EB)
    col = jnp.pad(edge_index[1], (0, _EPAD - _E),
                  constant_values=_N).reshape(_NW * _NB, _EB)
    zeros = jnp.zeros((_RPT, _D), jnp.float32)

    hist = _deg_call(col)
    ha = hist[:_NPAD].reshape(_NPAD, 1)
    hb = hist[_NPAD:].reshape(_NPAD, 1)

    y0, t0 = _tc_pre(xp, W0.T, B0.T, ha, hb)
    s0 = _scatter_call(y0, row, col, zeros)
    y1, t1 = _tc_mid(s0[0], s0[1], t0, ha, hb, W1.T, B1.T)
    s1 = _scatter_call(y1, row, col, zeros)
    out = _tc_fin(s1[0], s1[1], t1, ha, hb)
    return out[:_N]
